# baseline (speedup 1.0000x reference)
import functools

import jax
import jax.numpy as jnp
from jax import lax
from jax.experimental import pallas as pl
from jax.experimental.pallas import tpu as pltpu
from jax.experimental.pallas import tpu_sc as plsc

NC = 2
NS = 16
LANES = 16


def kernel(x, emb, posenc):
    B, T = x.shape
    V, D = emb.shape
    total = B * T
    NW = NC * NS
    per_w = total // NW
    C = 4 * T
    G = 100
    KG = C // G
    NCH = per_w // C
    KV = D // LANES

    xf = x.astype(jnp.int32).reshape(total // G, G)

    mesh = plsc.VectorSubcoreMesh(core_axis_name="c", subcore_axis_name="s")

    @functools.partial(
        pl.kernel,
        mesh=mesh,
        out_type=jax.ShapeDtypeStruct((total, D), jnp.float32),
        compiler_params=pltpu.CompilerParams(use_tc_tiling_on_sc=False),
        scratch_types=[
            pltpu.VMEM((KG, G), jnp.int32),
            pltpu.VMEM((C, D), jnp.float32),
            pltpu.VMEM((T, D), jnp.float32),
            pltpu.SemaphoreType.DMA,
        ],
    )
    def run(x_hbm, emb_hbm, pos_hbm, out_hbm, idx_v, rows_v, pos_v, sem):
        wid = lax.axis_index("s") * NC + lax.axis_index("c")
        pltpu.sync_copy(pos_hbm.at[pl.ds(0, T)], pos_v)

        def chunk_body(c, carry):
            base = wid * per_w + c * C
            row0 = wid * (per_w // G) + c * KG
            pltpu.sync_copy(x_hbm.at[pl.ds(row0, KG)], idx_v)
            cps = [
                pltpu.make_async_copy(
                    emb_hbm.at[idx_v.at[j]],
                    rows_v.at[pl.ds(j * G, G)],
                    sem,
                )
                for j in range(KG)
            ]
            for cp in cps:
                cp.start()
            for cp in cps:
                cp.wait()

            def t_body(t, carry2):
                pvec = [pos_v[t, pl.ds(k * LANES, LANES)] for k in range(KV)]
                for rep in range(C // T):
                    r = rep * T + t
                    for k in range(KV):
                        sl = pl.ds(k * LANES, LANES)
                        rows_v[r, sl] = rows_v[r, sl] + pvec[k]
                return carry2

            lax.fori_loop(0, T, t_body, 0)
            pltpu.sync_copy(rows_v, out_hbm.at[pl.ds(base, C)])
            return carry

        lax.fori_loop(0, NCH, chunk_body, 0)

    out = run(xf, emb, posenc)
    return out.reshape(B, T, D)


# speedup vs baseline: 1.0849x; 1.0849x over previous
import functools

import jax
import jax.numpy as jnp
from jax import lax
from jax.experimental import pallas as pl
from jax.experimental.pallas import tpu as pltpu
from jax.experimental.pallas import tpu_sc as plsc

NC = 2
NS = 16
LANES = 16


def kernel(x, emb, posenc):
    B, T = x.shape
    V, D = emb.shape
    total = B * T
    NW = NC * NS
    per_w = total // NW
    C = 4 * T
    G = 100
    KG = C // G
    NCH = per_w // C
    KV = D // LANES
    REP = C // T

    xf = x.astype(jnp.int32).reshape(total // G, G)

    mesh = plsc.VectorSubcoreMesh(core_axis_name="c", subcore_axis_name="s")

    @functools.partial(
        pl.kernel,
        mesh=mesh,
        out_type=jax.ShapeDtypeStruct((total, D), jnp.float32),
        compiler_params=pltpu.CompilerParams(use_tc_tiling_on_sc=False),
        scratch_types=[
            pltpu.VMEM((2, KG, G), jnp.int32),
            pltpu.VMEM((2, C, D), jnp.float32),
            pltpu.VMEM((T, D), jnp.float32),
            pltpu.SemaphoreType.DMA,
            pltpu.SemaphoreType.DMA,
            pltpu.SemaphoreType.DMA,
        ],
    )
    def run(x_hbm, emb_hbm, pos_hbm, out_hbm, idx_v, rows_v, pos_v,
            isem, gsem, wsem):
        wid = lax.axis_index("s") * NC + lax.axis_index("c")
        row0_w = wid * (per_w // G)
        base_w = wid * per_w
        pltpu.sync_copy(pos_hbm.at[pl.ds(0, T)], pos_v)

        def start_gathers(c, buf):
            for j in range(KG):
                pltpu.make_async_copy(
                    emb_hbm.at[idx_v.at[buf].at[j]],
                    rows_v.at[buf].at[pl.ds(j * G, G)],
                    gsem,
                ).start()

        def wait_gathers(buf):
            for j in range(KG):
                pltpu.make_async_copy(
                    emb_hbm.at[idx_v.at[buf].at[j]],
                    rows_v.at[buf].at[pl.ds(j * G, G)],
                    gsem,
                ).wait()

        def write_copy(c, buf):
            return pltpu.make_async_copy(
                rows_v.at[buf],
                out_hbm.at[pl.ds(base_w + c * C, C)],
                wsem,
            )

        pltpu.sync_copy(x_hbm.at[pl.ds(row0_w, KG)], idx_v.at[0])
        start_gathers(0, 0)

        def chunk_body(c, carry):
            b0 = lax.rem(c, 2)
            b1 = lax.rem(c + 1, 2)

            @pl.when(c + 1 < NCH)
            def _():
                pltpu.make_async_copy(
                    x_hbm.at[pl.ds(row0_w + (c + 1) * KG, KG)],
                    idx_v.at[b1],
                    isem,
                ).start()

            wait_gathers(b0)

            @pl.when(c + 1 < NCH)
            def _():
                pltpu.make_async_copy(
                    x_hbm.at[pl.ds(row0_w + (c + 1) * KG, KG)],
                    idx_v.at[b1],
                    isem,
                ).wait()

                @pl.when(c > 0)
                def _():
                    write_copy(c - 1, b1).wait()

                start_gathers(c + 1, b1)

            @plsc.parallel_loop(0, T, unroll=2)
            def t_body(t):
                pvec = [pos_v[t, pl.ds(k * LANES, LANES)] for k in range(KV)]
                for rep in range(REP):
                    r = rep * T + t
                    for k in range(KV):
                        sl = pl.ds(k * LANES, LANES)
                        rows_v[b0, r, sl] = rows_v[b0, r, sl] + pvec[k]

            write_copy(c, b0).start()
            return carry

        lax.fori_loop(0, NCH, chunk_body, 0)
        write_copy(NCH - 2, (NCH - 2) % 2).wait()
        write_copy(NCH - 1, (NCH - 1) % 2).wait()

    out = run(xf, emb, posenc)
    return out.reshape(B, T, D)
